# baseline (speedup 1.0000x reference)
import functools
import jax
import jax.numpy as jnp
from jax.experimental import pallas as pl
from jax.experimental.pallas import tpu as pltpu

QP = 64
C = 64
NPARAM = 11


def _body(idx_ref, p_ref, x_ref, o_ref):
    del idx_ref
    p = p_ref[0]

    def col(i):
        return p[:, i].reshape(1, C, 1)

    sp = [jax.nn.softplus(col(i)) for i in (0, 3, 6, 9)]
    ta = [jnp.tanh(col(i)) for i in (2, 5, 8)]
    bi = [col(i) for i in (1, 4, 7, 10)]

    y = x_ref[...]
    for layer in range(3):
        y = y * sp[layer] + bi[layer]
        y = y + jnp.tanh(y) * ta[layer]
    y = y * sp[3] + bi[3]
    o_ref[...] = jax.nn.sigmoid(y)


@jax.jit
def kernel(x, index, h1, b1, a1, h2, b2, a2, h3, b3, a3, h4, b4):
    B, Cx, H, W = x.shape
    HW = H * W
    xf = x.reshape(B, Cx, HW)
    table = jnp.stack(
        [t.reshape(QP, C) for t in (h1, b1, a1, h2, b2, a2, h3, b3, a3, h4, b4)],
        axis=2,
    )

    S = 1
    L = HW // S
    grid_spec = pltpu.PrefetchScalarGridSpec(
        num_scalar_prefetch=1,
        grid=(B, S),
        in_specs=[
            pl.BlockSpec((1, C, NPARAM), lambda b, s, idx: (idx[b], 0, 0)),
            pl.BlockSpec((1, Cx, L), lambda b, s, idx: (b, 0, s)),
        ],
        out_specs=pl.BlockSpec((1, Cx, L), lambda b, s, idx: (b, 0, s)),
    )
    out = pl.pallas_call(
        _body,
        grid_spec=grid_spec,
        out_shape=jax.ShapeDtypeStruct((B, Cx, HW), x.dtype),
    )(index, table, xf)
    return out.reshape(B, Cx, H, W)
